# baseline (speedup 1.0000x reference)
import functools

import jax
import jax.numpy as jnp
from jax import lax
from jax.experimental import pallas as pl
from jax.experimental.pallas import tpu as pltpu
from jax.experimental.pallas import tpu_sc as plsc

_VOCAB = 100000
_D = 64
_B = 4096
_C = 20
_LANES = 16

_NC = 2
_NS = 16
_NW = _NC * _NS
_BPW = _B // _NW
_CB = 4
_NCHUNK = _BPW // _CB
_IDXC = _CB * _C
_NBUF = 2


def _cbow_body(idx_hbm, table_hbm, out_hbm, idx_v, bufs, out_v, sems):
    wid = lax.axis_index("s") * _NC + lax.axis_index("c")
    base = wid * _BPW
    inv_c = jnp.float32(1.0 / _C)

    pltpu.sync_copy(idx_hbm.at[pl.ds(base * _C, _BPW * _C)], idx_v)

    def _gather(chunk, b):
        return pltpu.make_async_copy(
            table_hbm.at[idx_v.at[pl.ds(chunk * _IDXC, _IDXC)]],
            bufs[b], sems[b])

    for b in range(_NBUF):
        _gather(b, b).start()

    @pl.loop(0, _NCHUNK, step=_NBUF, unroll=1)
    def _chunk_loop(j):
        for b in range(_NBUF):
            chunk = j + b
            _gather(chunk, b).wait()
            buf = bufs[b]
            for r in range(_CB):
                row = r * _C
                for k in range(_D // _LANES):
                    sl = pl.ds(k * _LANES, _LANES)
                    acc = buf[row, sl]
                    for c in range(1, _C):
                        acc = acc + buf[row + c, sl]
                    out_v[chunk * _CB + r, sl] = acc * inv_c

            @pl.when(chunk + _NBUF < _NCHUNK)
            def _():
                _gather(chunk + _NBUF, b).start()

    pltpu.sync_copy(out_v, out_hbm.at[pl.ds(base, _BPW)])


@jax.jit
def _cbow_sc(idx, table):
    mesh = plsc.VectorSubcoreMesh(
        core_axis_name="c", subcore_axis_name="s",
        num_cores=_NC, num_subcores=_NS)
    f = pl.kernel(
        _cbow_body,
        out_type=jax.ShapeDtypeStruct((_B, _D), jnp.float32),
        mesh=mesh,
        scratch_types=[
            pltpu.VMEM((_BPW * _C,), jnp.int32),
            tuple(pltpu.VMEM((_IDXC, _D), jnp.float32)
                  for _ in range(_NBUF)),
            pltpu.VMEM((_BPW, _D), jnp.float32),
            tuple(pltpu.SemaphoreType.DMA for _ in range(_NBUF)),
        ],
        compiler_params=pltpu.CompilerParams(use_tc_tiling_on_sc=False),
    )
    return f(idx, table)


def kernel(context_indices, in_embed):
    idx = context_indices.astype(jnp.int32).reshape(-1)
    return _cbow_sc(idx, in_embed)


# speedup vs baseline: 2.0511x; 2.0511x over previous
import jax
import jax.numpy as jnp
from jax import lax
from jax.experimental import pallas as pl
from jax.experimental.pallas import tpu as pltpu
from jax.experimental.pallas import tpu_sc as plsc

_VOCAB = 100000
_D = 64
_B = 4096
_C = 20
_L = 16

_NC = 2
_NS = 16
_NW = _NC * _NS
_ROUNDS = _D // _NW
_BCHUNK = 512
_NBCHUNK = _B // _BCHUNK
_CHUNKW = _C * _BCHUNK


def _cbow_body(idx_hbm, table_hbm, out_hbm, row_v, idxbufs, outr_v,
               row_sem, idx_sems):
    wid = lax.axis_index("s") * _NC + lax.axis_index("c")
    inv_c = jnp.float32(1.0 / _C)

    def _idx_copy(chunk, b):
        return pltpu.make_async_copy(
            idx_hbm.at[pl.ds(chunk * _CHUNKW, _CHUNKW)],
            idxbufs[b], idx_sems[b])

    for r in range(_ROUNDS):
        d = wid + r * _NW
        row_copy = pltpu.make_async_copy(table_hbm.at[d], row_v, row_sem)
        row_copy.start()
        _idx_copy(0, 0).start()
        row_copy.wait()

        @pl.loop(0, _NBCHUNK, step=2, unroll=1)
        def _chunk_loop(j):
            for b in range(2):
                chunk = j + b
                _idx_copy(chunk, b).wait()

                @pl.when(chunk + 1 < _NBCHUNK)
                def _():
                    _idx_copy(chunk + 1, 1 - b).start()

                idxb_v = idxbufs[b]

                @plsc.parallel_loop(0, _BCHUNK // _L)
                def _vec_loop(bb):
                    off = bb * _L
                    acc0 = plsc.load_gather(
                        row_v, [idxb_v[pl.ds(off, _L)]])
                    acc1 = plsc.load_gather(
                        row_v, [idxb_v[pl.ds(_BCHUNK + off, _L)]])
                    for c in range(2, _C, 2):
                        acc0 = acc0 + plsc.load_gather(
                            row_v, [idxb_v[pl.ds(c * _BCHUNK + off, _L)]])
                        acc1 = acc1 + plsc.load_gather(
                            row_v,
                            [idxb_v[pl.ds((c + 1) * _BCHUNK + off, _L)]])
                    outr_v[pl.ds(chunk * _BCHUNK + off, _L)] = (
                        (acc0 + acc1) * inv_c)

        pltpu.sync_copy(outr_v, out_hbm.at[d])


@jax.jit
def _cbow_sc(idx_flat, table_t):
    mesh = plsc.VectorSubcoreMesh(
        core_axis_name="c", subcore_axis_name="s",
        num_cores=_NC, num_subcores=_NS)
    f = pl.kernel(
        _cbow_body,
        out_type=jax.ShapeDtypeStruct((_D, _B), jnp.float32),
        mesh=mesh,
        scratch_types=[
            pltpu.VMEM((_VOCAB,), jnp.float32),
            tuple(pltpu.VMEM((_CHUNKW,), jnp.int32) for _ in range(2)),
            pltpu.VMEM((_B,), jnp.float32),
            pltpu.SemaphoreType.DMA,
            tuple(pltpu.SemaphoreType.DMA for _ in range(2)),
        ],
        compiler_params=pltpu.CompilerParams(
            use_tc_tiling_on_sc=True, needs_layout_passes=False),
    )
    return f(idx_flat, table_t)


def kernel(context_indices, in_embed):
    idx_flat = (context_indices.astype(jnp.int32).T
                .reshape(_C, _NBCHUNK, _BCHUNK)
                .transpose(1, 0, 2).reshape(-1))
    table_t = in_embed.T
    out_t = _cbow_sc(idx_flat, table_t)
    return out_t.T


# speedup vs baseline: 2.0579x; 1.0033x over previous
import jax
import jax.numpy as jnp
from jax import lax
from jax.experimental import pallas as pl
from jax.experimental.pallas import tpu as pltpu
from jax.experimental.pallas import tpu_sc as plsc

_VOCAB = 100000
_D = 64
_B = 4096
_C = 20
_L = 16

_NC = 2
_NS = 16
_NW = _NC * _NS
_ROUNDS = _D // _NW
_BCHUNK = 512
_NBCHUNK = _B // _BCHUNK
_CHUNKW = _C * _BCHUNK


def _cbow_body(idx_hbm, table_hbm, out_hbm, row_v, idxbufs, outr_v,
               row_sem, idx_sems):
    wid = lax.axis_index("s") * _NC + lax.axis_index("c")
    inv_c = jnp.float32(1.0 / _C)

    def _idx_copy(chunk, b):
        return pltpu.make_async_copy(
            idx_hbm.at[pl.ds(chunk * _CHUNKW, _CHUNKW)],
            idxbufs[b], idx_sems[b])

    for r in range(_ROUNDS):
        d = wid + r * _NW
        row_copy = pltpu.make_async_copy(table_hbm.at[d], row_v, row_sem)
        row_copy.start()
        _idx_copy(0, 0).start()
        row_copy.wait()

        @pl.loop(0, _NBCHUNK, step=2, unroll=1)
        def _chunk_loop(j):
            for b in range(2):
                chunk = j + b
                _idx_copy(chunk, b).wait()

                @pl.when(chunk + 1 < _NBCHUNK)
                def _():
                    _idx_copy(chunk + 1, 1 - b).start()

                idxb_v = idxbufs[b]

                @plsc.parallel_loop(0, _BCHUNK // _L)
                def _vec_loop(bb):
                    off = bb * _L
                    acc0 = plsc.load_gather(
                        row_v, [idxb_v[pl.ds(off, _L)]])
                    acc1 = plsc.load_gather(
                        row_v, [idxb_v[pl.ds(_BCHUNK + off, _L)]])
                    for c in range(2, _C, 2):
                        acc0 = acc0 + plsc.load_gather(
                            row_v, [idxb_v[pl.ds(c * _BCHUNK + off, _L)]])
                        acc1 = acc1 + plsc.load_gather(
                            row_v,
                            [idxb_v[pl.ds((c + 1) * _BCHUNK + off, _L)]])
                    outr_v[pl.ds(chunk * _BCHUNK + off, _L)] = (
                        (acc0 + acc1) * inv_c)

        pltpu.sync_copy(outr_v, out_hbm.at[d])


@jax.jit
def _cbow_sc(idx_flat, table_t):
    mesh = plsc.VectorSubcoreMesh(
        core_axis_name="c", subcore_axis_name="s",
        num_cores=_NC, num_subcores=_NS)
    f = pl.kernel(
        _cbow_body,
        out_type=jax.ShapeDtypeStruct((_D, _B), jnp.float32),
        mesh=mesh,
        scratch_types=[
            pltpu.VMEM((_VOCAB,), jnp.float32),
            tuple(pltpu.VMEM((_CHUNKW,), jnp.int32) for _ in range(2)),
            pltpu.VMEM((_B,), jnp.float32),
            pltpu.SemaphoreType.DMA,
            tuple(pltpu.SemaphoreType.DMA for _ in range(2)),
        ],
        compiler_params=pltpu.CompilerParams(
            use_tc_tiling_on_sc=True, needs_layout_passes=False,
            skip_device_barrier=True),
    )
    return f(idx_flat, table_t)


def kernel(context_indices, in_embed):
    idx_flat = (context_indices.astype(jnp.int32).T
                .reshape(_C, _NBCHUNK, _BCHUNK)
                .transpose(1, 0, 2).reshape(-1))
    table_t = in_embed.T
    out_t = _cbow_sc(idx_flat, table_t)
    return out_t.T


# speedup vs baseline: 2.3648x; 1.1491x over previous
import jax
import jax.numpy as jnp
from jax import lax
from jax.experimental import pallas as pl
from jax.experimental.pallas import tpu as pltpu
from jax.experimental.pallas import tpu_sc as plsc

_VOCAB = 100000
_D = 64
_B = 4096
_C = 20
_L = 16

_NC = 2
_NS = 16
_NW = _NC * _NS
_ROUNDS = _D // _NW
_BCHUNK = 512
_NBCHUNK = _B // _BCHUNK
_CHUNKW = _C * _BCHUNK


def _cbow_body(idx_hbm, table_hbm, out_hbm, row_v, idxbufs, outr_v,
               idx_shared, row_sem, idx_sems):
    wid = lax.axis_index("s") * _NC + lax.axis_index("c")
    sid = lax.axis_index("s")
    inv_c = jnp.float32(1.0 / _C)

    @pl.when(sid == 0)
    def _():
        pltpu.sync_copy(idx_hbm, idx_shared)

    plsc.subcore_barrier()

    def _idx_copy(chunk, b):
        return pltpu.make_async_copy(
            idx_shared.at[pl.ds(chunk * _CHUNKW, _CHUNKW)],
            idxbufs[b], idx_sems[b])

    for r in range(_ROUNDS):
        d = wid + r * _NW
        row_copy = pltpu.make_async_copy(table_hbm.at[d], row_v, row_sem)
        row_copy.start()
        _idx_copy(0, 0).start()
        row_copy.wait()

        @pl.loop(0, _NBCHUNK, step=2, unroll=1)
        def _chunk_loop(j):
            for b in range(2):
                chunk = j + b
                _idx_copy(chunk, b).wait()

                @pl.when(chunk + 1 < _NBCHUNK)
                def _():
                    _idx_copy(chunk + 1, 1 - b).start()

                idxb_v = idxbufs[b]

                @plsc.parallel_loop(0, _BCHUNK // _L)
                def _vec_loop(bb):
                    off = bb * _L
                    acc0 = plsc.load_gather(
                        row_v, [idxb_v[pl.ds(off, _L)]])
                    acc1 = plsc.load_gather(
                        row_v, [idxb_v[pl.ds(_BCHUNK + off, _L)]])
                    for c in range(2, _C, 2):
                        acc0 = acc0 + plsc.load_gather(
                            row_v, [idxb_v[pl.ds(c * _BCHUNK + off, _L)]])
                        acc1 = acc1 + plsc.load_gather(
                            row_v,
                            [idxb_v[pl.ds((c + 1) * _BCHUNK + off, _L)]])
                    outr_v[pl.ds(chunk * _BCHUNK + off, _L)] = (
                        (acc0 + acc1) * inv_c)

        pltpu.sync_copy(outr_v, out_hbm.at[d])


@jax.jit
def _cbow_sc(idx_flat, table_t):
    mesh = plsc.VectorSubcoreMesh(
        core_axis_name="c", subcore_axis_name="s",
        num_cores=_NC, num_subcores=_NS)
    f = pl.kernel(
        _cbow_body,
        out_type=jax.ShapeDtypeStruct((_D, _B), jnp.float32),
        mesh=mesh,
        scratch_types=[
            pltpu.VMEM((_VOCAB,), jnp.float32),
            tuple(pltpu.VMEM((_CHUNKW,), jnp.int32) for _ in range(2)),
            pltpu.VMEM((_B,), jnp.float32),
            pltpu.VMEM_SHARED((_B * _C,), jnp.int32),
            pltpu.SemaphoreType.DMA,
            tuple(pltpu.SemaphoreType.DMA for _ in range(2)),
        ],
        compiler_params=pltpu.CompilerParams(
            use_tc_tiling_on_sc=True, needs_layout_passes=False),
    )
    return f(idx_flat, table_t)


def kernel(context_indices, in_embed):
    idx_flat = (context_indices.astype(jnp.int32).T
                .reshape(_C, _NBCHUNK, _BCHUNK)
                .transpose(1, 0, 2).reshape(-1))
    table_t = in_embed.T
    out_t = _cbow_sc(idx_flat, table_t)
    return out_t.T
